# Initial kernel scaffold; baseline (speedup 1.0000x reference)
import jax
import jax.numpy as jnp
from jax.experimental import pallas as pl


def kernel(inputs, position_embeddings):
    raise NotImplementedError("write your pallas kernel here")


# speedup vs baseline: 1.6826x; 1.6826x over previous
import jax
import jax.numpy as jnp
from jax.experimental import pallas as pl


def _add_kernel(x_ref, p_ref, o_ref):
    o_ref[...] = x_ref[...] + p_ref[...][None, :, :]


def kernel(inputs, position_embeddings):
    B, S, D = inputs.shape
    S_BLK = 512
    grid = (S // S_BLK, B)
    return pl.pallas_call(
        _add_kernel,
        grid=grid,
        in_specs=[
            pl.BlockSpec((1, S_BLK, D), lambda i, b: (b, i, 0)),
            pl.BlockSpec((S_BLK, D), lambda i, b: (i, 0)),
        ],
        out_specs=pl.BlockSpec((1, S_BLK, D), lambda i, b: (b, i, 0)),
        out_shape=jax.ShapeDtypeStruct((B, S, D), inputs.dtype),
    )(inputs, position_embeddings)


# speedup vs baseline: 1.8769x; 1.1154x over previous
import jax
import jax.numpy as jnp
from jax.experimental import pallas as pl


def _add_kernel(x_ref, p_ref, o_ref):
    o_ref[...] = x_ref[...] + p_ref[...][None, :, :]


def kernel(inputs, position_embeddings):
    B, S, D = inputs.shape
    S_BLK = 1024
    grid = (S // S_BLK, B)
    return pl.pallas_call(
        _add_kernel,
        grid=grid,
        in_specs=[
            pl.BlockSpec((1, S_BLK, D), lambda i, b: (b, i, 0)),
            pl.BlockSpec((S_BLK, D), lambda i, b: (i, 0)),
        ],
        out_specs=pl.BlockSpec((1, S_BLK, D), lambda i, b: (b, i, 0)),
        out_shape=jax.ShapeDtypeStruct((B, S, D), inputs.dtype),
    )(inputs, position_embeddings)


# speedup vs baseline: 1.9883x; 1.0594x over previous
import jax
import jax.numpy as jnp
from jax.experimental import pallas as pl


def _add_kernel(x_ref, p_ref, o_ref):
    o_ref[...] = x_ref[...] + p_ref[...][None, :, :]


def kernel(inputs, position_embeddings):
    B, S, D = inputs.shape
    S_BLK = 2048
    grid = (S // S_BLK, B)
    return pl.pallas_call(
        _add_kernel,
        grid=grid,
        in_specs=[
            pl.BlockSpec((1, S_BLK, D), lambda i, b: (b, i, 0)),
            pl.BlockSpec((S_BLK, D), lambda i, b: (i, 0)),
        ],
        out_specs=pl.BlockSpec((1, S_BLK, D), lambda i, b: (b, i, 0)),
        out_shape=jax.ShapeDtypeStruct((B, S, D), inputs.dtype),
    )(inputs, position_embeddings)
